# baseline (speedup 1.0000x reference)
import jax
import jax.numpy as jnp
from jax import lax
from jax.experimental import pallas as pl
from jax.experimental.pallas import tpu as pltpu
from jax.experimental.pallas import tpu_sc as plsc

_B = 16384
_D = 64
_NE = 1000000
_NR = 1000
_NC = 2
_NS = 16
_NW = _NC * _NS
_BPW = _B // _NW
_HALF = _BPW // 2
_CHUNK = 128

_NSLAB = _NE // 128
_SPW = (_NSLAB + _NW - 1) // _NW


def _format_body(entT_hbm, tail_hbm, out_hbm, in2, out2, tailb, sem_in, sem_out):
    wid = lax.axis_index("s") * _NC + lax.axis_index("c")

    lanes16 = lax.iota(jnp.int32, 16)
    dimsel = [jnp.full((16,), (q * 16) % _D, jnp.int32) + lanes16
              for q in range(8)]

    def slab_of(step):
        return jnp.minimum(step * _NW + wid, _NSLAB - 1)

    def fire_in(step, slot):
        j = slab_of(step)
        return pltpu.async_copy(
            entT_hbm.at[:, pl.ds(j * 128, 128)], in2.at[slot], sem_in)

    def fire_out(step, slot):
        j = slab_of(step)
        return pltpu.async_copy(
            out2.at[slot], out_hbm.at[pl.ds(j * 64, 64), :], sem_out)

    def drain_in(slot):
        pltpu.make_async_copy(
            entT_hbm.at[:, pl.ds(0, 128)], in2.at[slot], sem_in).wait()

    def drain_out(slot):
        pltpu.make_async_copy(
            out2.at[slot], out_hbm.at[pl.ds(0, 64), :], sem_out).wait()

    def transpose(slot):
        for p in range(64):
            for q in range(8):
                e = jnp.full((16,), 2 * p + (1 if q >= 4 else 0), jnp.int32)
                x = plsc.load_gather(in2.at[slot], [dimsel[q], e])
                out2[slot, p, pl.ds(q * 16, 16)] = x

    fire_in(0, 0)

    def step(i, carry):
        a = i * 2
        fire_in(a + 1, 1)
        drain_in(0)
        @pl.when(i > 0)
        def _():
            drain_out(0)
        transpose(0)
        fire_out(a, 0)
        fire_in(jnp.minimum(a + 2, _SPW - 1), 0)
        drain_in(1)
        @pl.when(i > 0)
        def _():
            drain_out(1)
        transpose(1)
        fire_out(a + 1, 1)
        return carry

    lax.fori_loop(0, _SPW // 2, step, 0)
    last = _SPW - 1
    drain_in(0)
    drain_out(0)
    transpose(0)
    fire_out(last, 0)
    drain_out(1)
    drain_out(0)

    @pl.when(wid == 0)
    def _tail():
        pltpu.async_copy(tail_hbm, tailb, sem_in).wait()
        pltpu.async_copy(
            tailb, out_hbm.at[pl.ds(_NSLAB * 64, 32), :], sem_out).wait()


def _score_body(ent_hbm, rel_hbm, heads_hbm, rels_hbm, tails_hbm, out_hbm,
                hidx, ridx, tidx, hp, rp, tp, e1_v, r_v, e2_v, out_v, sem):
    wid = lax.axis_index("s") * _NC + lax.axis_index("c")
    base = wid * _BPW

    pltpu.sync_copy(heads_hbm.at[pl.ds(base, _BPW)], hidx)
    pltpu.sync_copy(rels_hbm.at[pl.ds(base, _BPW)], ridx)
    pltpu.sync_copy(tails_hbm.at[pl.ds(base, _BPW)], tidx)

    for k in range(_BPW // 16):
        sl = pl.ds(k * 16, 16)
        hp[sl] = hidx[sl] >> 1
        rp[sl] = ridx[sl] >> 1
        tp[sl] = tidx[sl] >> 1

    lanes16 = lax.iota(jnp.int32, 16)
    bfly = [jnp.bitwise_xor(lanes16, sh) for sh in (8, 4, 2, 1)]
    dnums = lax.GatherDimensionNumbers(
        offset_dims=(), collapsed_slice_dims=(0,), start_index_map=(0,))

    def shuffle(v, idx):
        return lax.gather(v, idx[:, None], dnums, slice_sizes=(1,),
                          mode=lax.GatherScatterMode.PROMISE_IN_BOUNDS)

    def lanesum(v):
        for idx in bfly:
            v = v + shuffle(v, idx)
        return v

    ones16 = jnp.ones((16,), jnp.int32)

    for half in range(2):
        hbase = half * _HALF
        copies = []
        for k in range(_HALF // _CHUNK):
            isl = pl.ds(hbase + k * _CHUNK, _CHUNK)
            vsl = pl.ds(k * _CHUNK, _CHUNK)
            copies.append(pltpu.async_copy(ent_hbm.at[hp.at[isl]], e1_v.at[vsl], sem))
            copies.append(pltpu.async_copy(rel_hbm.at[rp.at[isl]], r_v.at[vsl], sem))
            copies.append(pltpu.async_copy(ent_hbm.at[tp.at[isl]], e2_v.at[vsl], sem))
        for c in copies:
            c.wait()

        def group(g, carry):
            row0 = g * 16
            hparf = (hidx[pl.ds(hbase + row0, 16)] & ones16).astype(jnp.float32)
            rparf = (ridx[pl.ds(hbase + row0, 16)] & ones16).astype(jnp.float32)
            tparf = (tidx[pl.ds(hbase + row0, 16)] & ones16).astype(jnp.float32)
            s = jnp.zeros((16,), jnp.float32)
            for j in range(16):
                row = row0 + j
                jv = jnp.full((16,), j, jnp.int32)
                ph = shuffle(hparf, jv)
                pr = shuffle(rparf, jv)
                pt = shuffle(tparf, jv)
                acc = jnp.zeros((16,), jnp.float32)
                for c in range(_D // 16):
                    lo = pl.ds(c * 16, 16)
                    hi = pl.ds(64 + c * 16, 16)
                    a1 = e1_v[row, lo]
                    a = a1 + ph * (e1_v[row, hi] - a1)
                    b1 = r_v[row, lo]
                    b = b1 + pr * (r_v[row, hi] - b1)
                    d1 = e2_v[row, lo]
                    d2 = d1 + pt * (e2_v[row, hi] - d1)
                    acc = acc + (a * b) * d2
                s = jnp.where(lanes16 == j, lanesum(acc), s)
            out_v[pl.ds(row0, 16)] = 1.0 / (1.0 + jnp.exp(-s))
            return carry

        lax.fori_loop(0, _HALF // 16, group, 0)
        pltpu.sync_copy(out_v.at[pl.ds(0, _HALF)],
                        out_hbm.at[pl.ds(base + hbase, _HALF)])


def kernel(entity_embedding, relation_embedding, heads, relations, tails):
    mesh = plsc.VectorSubcoreMesh(core_axis_name="c", subcore_axis_name="s")

    fmt = pl.kernel(
        _format_body,
        out_type=jax.ShapeDtypeStruct((_NE // 2, 128), jnp.float32),
        mesh=mesh,
        compiler_params=pltpu.CompilerParams(needs_layout_passes=False),
        scratch_types=[
            pltpu.VMEM((2, _D, 128), jnp.float32),
            pltpu.VMEM((2, _D, 128), jnp.float32),
            pltpu.VMEM((32, 128), jnp.float32),
            pltpu.SemaphoreType.DMA,
            pltpu.SemaphoreType.DMA,
        ],
    )
    ent2 = fmt(entity_embedding.T,
               entity_embedding[_NSLAB * 128:].reshape(32, 128))

    score = pl.kernel(
        _score_body,
        out_type=jax.ShapeDtypeStruct((_B,), jnp.float32),
        mesh=mesh,
        scratch_types=[
            pltpu.VMEM((_BPW,), jnp.int32),
            pltpu.VMEM((_BPW,), jnp.int32),
            pltpu.VMEM((_BPW,), jnp.int32),
            pltpu.VMEM((_BPW,), jnp.int32),
            pltpu.VMEM((_BPW,), jnp.int32),
            pltpu.VMEM((_BPW,), jnp.int32),
            pltpu.VMEM((_HALF, 2 * _D), jnp.float32),
            pltpu.VMEM((_HALF, 2 * _D), jnp.float32),
            pltpu.VMEM((_HALF, 2 * _D), jnp.float32),
            pltpu.VMEM((_HALF,), jnp.float32),
            pltpu.SemaphoreType.DMA,
        ],
    )
    return score(ent2, relation_embedding.reshape(_NR // 2, 128),
                 heads.astype(jnp.int32), relations.astype(jnp.int32),
                 tails.astype(jnp.int32))


# speedup vs baseline: 4.2536x; 4.2536x over previous
import jax
import jax.numpy as jnp
from jax import lax
from jax.experimental import pallas as pl
from jax.experimental.pallas import tpu as pltpu
from jax.experimental.pallas import tpu_sc as plsc

_B = 16384
_D = 64
_NR = 1000
_NC = 2
_NS = 16
_NW = _NC * _NS
_BPW = _B // _NW
_HALF = _BPW // 2
_CH = 16
_NCH = _HALF // _CH


def _fire(ent_hbm, hvec, tvec, e1b, e2b, sem):
    for j in range(_CH):
        h8 = pl.multiple_of((hvec[j] >> 3) * 8, 8)
        t8 = pl.multiple_of((tvec[j] >> 3) * 8, 8)
        pltpu.async_copy(ent_hbm.at[pl.ds(h8, 8), :], e1b.at[j], sem)
        pltpu.async_copy(ent_hbm.at[pl.ds(t8, 8), :], e2b.at[j], sem)


def _body(ent_hbm, rel_hbm, heads_hbm, rels_hbm, tails_hbm, out_hbm,
          hidx, ridx, tidx, rp, e1b2, e2b2, r_v, out_v, sem, rsem):
    wid = lax.axis_index("s") * _NC + lax.axis_index("c")
    base = wid * _BPW

    pltpu.sync_copy(heads_hbm.at[pl.ds(base, _BPW)], hidx)
    pltpu.sync_copy(rels_hbm.at[pl.ds(base, _BPW)], ridx)
    pltpu.sync_copy(tails_hbm.at[pl.ds(base, _BPW)], tidx)

    for k in range(_BPW // 16):
        sl = pl.ds(k * 16, 16)
        rp[sl] = ridx[sl] >> 1

    lanes16 = lax.iota(jnp.int32, 16)
    ones16 = jnp.ones((16,), jnp.int32)
    bfly = [jnp.bitwise_xor(lanes16, sh) for sh in (8, 4, 2, 1)]
    dnums = lax.GatherDimensionNumbers(
        offset_dims=(), collapsed_slice_dims=(0,), start_index_map=(0,))

    def shuffle(v, idx):
        return lax.gather(v, idx[:, None], dnums, slice_sizes=(1,),
                          mode=lax.GatherScatterMode.PROMISE_IN_BOUNDS)

    def lanesum(v):
        for idx in bfly:
            v = v + shuffle(v, idx)
        return v

    def idx_chunk(half, k):
        sl = pl.ds(half * _HALF + k * _CH, _CH)
        return hidx[sl], tidx[sl]

    def drain(slot):
        dummy = ent_hbm.at[pl.ds(0, 8), :]
        for j in range(_CH):
            pltpu.make_async_copy(dummy, e1b2.at[slot, j], sem).wait()
            pltpu.make_async_copy(dummy, e2b2.at[slot, j], sem).wait()

    def fire_chunk(half, k, slot):
        hv, tv = idx_chunk(half, k)
        _fire(ent_hbm, hv, tv, e1b2.at[slot], e2b2.at[slot], sem)

    def compute_chunk(half, k, slot):
        hvec, tvec = idx_chunk(half, k)
        sl = pl.ds(half * _HALF + k * _CH, _CH)
        rparf = (ridx[sl] & ones16).astype(jnp.float32)
        s = jnp.zeros((16,), jnp.float32)
        for j in range(_CH):
            hs = hvec[j] & 7
            ts = tvec[j] & 7
            row = k * _CH + j
            jv = jnp.full((16,), j, jnp.int32)
            pr = shuffle(rparf, jv)
            acc = jnp.zeros((16,), jnp.float32)
            for c in range(_D // 16):
                lo = pl.ds(c * 16, 16)
                hi = pl.ds(64 + c * 16, 16)
                b1 = r_v[row, lo]
                b = b1 + pr * (r_v[row, hi] - b1)
                acc = acc + (e1b2[slot, j, hs, lo] * b) \
                    * e2b2[slot, j, ts, lo]
            s = jnp.where(lanes16 == j, lanesum(acc), s)
        out_v[pl.ds(k * _CH, _CH)] = 1.0 / (1.0 + jnp.exp(-s))

    for half in range(2):
        rcopies = []
        for q in range(_HALF // 128):
            isl = pl.ds(half * _HALF + q * 128, 128)
            vsl = pl.ds(q * 128, 128)
            rcopies.append(
                pltpu.async_copy(rel_hbm.at[rp.at[isl]], r_v.at[vsl], rsem))

        fire_chunk(half, 0, 0)

        def step(i, carry):
            a = i * 2
            fire_chunk(half, a + 1, 1)
            drain(0)
            compute_chunk(half, a, 0)
            fire_chunk(half, jnp.minimum(a + 2, _NCH - 1), 0)
            drain(1)
            compute_chunk(half, a + 1, 1)
            return carry

        for c in rcopies:
            c.wait()
        lax.fori_loop(0, _NCH // 2, step, 0)
        drain(0)
        pltpu.sync_copy(out_v, out_hbm.at[pl.ds(base + half * _HALF, _HALF)])


def kernel(entity_embedding, relation_embedding, heads, relations, tails):
    mesh = plsc.VectorSubcoreMesh(core_axis_name="c", subcore_axis_name="s")
    run = pl.kernel(
        _body,
        out_type=jax.ShapeDtypeStruct((_B,), jnp.float32),
        mesh=mesh,
        scratch_types=[
            pltpu.VMEM((_BPW,), jnp.int32),
            pltpu.VMEM((_BPW,), jnp.int32),
            pltpu.VMEM((_BPW,), jnp.int32),
            pltpu.VMEM((_BPW,), jnp.int32),
            pltpu.VMEM((2, _CH, 8, _D), jnp.float32),
            pltpu.VMEM((2, _CH, 8, _D), jnp.float32),
            pltpu.VMEM((_HALF, 2 * _D), jnp.float32),
            pltpu.VMEM((_HALF,), jnp.float32),
            pltpu.SemaphoreType.DMA,
            pltpu.SemaphoreType.DMA,
        ],
    )
    return run(entity_embedding, relation_embedding.reshape(_NR // 2, 128),
               heads.astype(jnp.int32), relations.astype(jnp.int32),
               tails.astype(jnp.int32))


# speedup vs baseline: 4.5597x; 1.0720x over previous
import jax
import jax.numpy as jnp
from jax import lax
from jax.experimental import pallas as pl
from jax.experimental.pallas import tpu as pltpu
from jax.experimental.pallas import tpu_sc as plsc

_B = 16384
_D = 64
_NR = 1000
_NC = 2
_NS = 16
_NW = _NC * _NS
_BPW = _B // _NW
_HALF = _BPW // 2
_CH = 16
_NCH = _HALF // _CH


def _fire(ent_hbm, hvec, tvec, e1b, e2b, sem):
    for j in range(_CH):
        pltpu.async_copy(ent_hbm.at[pl.ds(hvec[j], 1), :], e1b.at[j], sem)
        pltpu.async_copy(ent_hbm.at[pl.ds(tvec[j], 1), :], e2b.at[j], sem)


def _body(ent_hbm, rel_hbm, heads_hbm, rels_hbm, tails_hbm, out_hbm,
          hidx, ridx, tidx, rp, e1b2, e2b2, r_v, out_v, sem, rsem):
    wid = lax.axis_index("s") * _NC + lax.axis_index("c")
    base = wid * _BPW

    pltpu.sync_copy(heads_hbm.at[pl.ds(base, _BPW)], hidx)
    pltpu.sync_copy(rels_hbm.at[pl.ds(base, _BPW)], ridx)
    pltpu.sync_copy(tails_hbm.at[pl.ds(base, _BPW)], tidx)

    for k in range(_BPW // 16):
        sl = pl.ds(k * 16, 16)
        rp[sl] = ridx[sl] >> 1

    lanes16 = lax.iota(jnp.int32, 16)
    ones16 = jnp.ones((16,), jnp.int32)
    bfly = [jnp.bitwise_xor(lanes16, sh) for sh in (8, 4, 2, 1)]
    dnums = lax.GatherDimensionNumbers(
        offset_dims=(), collapsed_slice_dims=(0,), start_index_map=(0,))

    def shuffle(v, idx):
        return lax.gather(v, idx[:, None], dnums, slice_sizes=(1,),
                          mode=lax.GatherScatterMode.PROMISE_IN_BOUNDS)

    def lanesum(v):
        for idx in bfly:
            v = v + shuffle(v, idx)
        return v

    def idx_chunk(half, k):
        sl = pl.ds(half * _HALF + k * _CH, _CH)
        return hidx[sl], tidx[sl]

    def drain(slot):
        dummy = ent_hbm.at[pl.ds(0, 1), :]
        for j in range(_CH):
            pltpu.make_async_copy(dummy, e1b2.at[slot, j], sem).wait()
            pltpu.make_async_copy(dummy, e2b2.at[slot, j], sem).wait()

    def fire_chunk(half, k, slot):
        hv, tv = idx_chunk(half, k)
        _fire(ent_hbm, hv, tv, e1b2.at[slot], e2b2.at[slot], sem)

    def compute_chunk(half, k, slot):
        sl = pl.ds(half * _HALF + k * _CH, _CH)
        rparf = (ridx[sl] & ones16).astype(jnp.float32)
        s = jnp.zeros((16,), jnp.float32)
        for j in range(_CH):
            row = k * _CH + j
            jv = jnp.full((16,), j, jnp.int32)
            pr = shuffle(rparf, jv)
            acc = jnp.zeros((16,), jnp.float32)
            for c in range(_D // 16):
                lo = pl.ds(c * 16, 16)
                hi = pl.ds(64 + c * 16, 16)
                b1 = r_v[row, lo]
                b = b1 + pr * (r_v[row, hi] - b1)
                acc = acc + (e1b2[slot, j, 0, lo] * b) \
                    * e2b2[slot, j, 0, lo]
            s = jnp.where(lanes16 == j, lanesum(acc), s)
        out_v[pl.ds(k * _CH, _CH)] = 1.0 / (1.0 + jnp.exp(-s))

    for half in range(2):
        rcopies = []
        for q in range(_HALF // 128):
            isl = pl.ds(half * _HALF + q * 128, 128)
            vsl = pl.ds(q * 128, 128)
            rcopies.append(
                pltpu.async_copy(rel_hbm.at[rp.at[isl]], r_v.at[vsl], rsem))

        fire_chunk(half, 0, 0)

        def step(i, carry):
            a = i * 2
            fire_chunk(half, a + 1, 1)
            drain(0)
            compute_chunk(half, a, 0)
            fire_chunk(half, jnp.minimum(a + 2, _NCH - 1), 0)
            drain(1)
            compute_chunk(half, a + 1, 1)
            return carry

        for c in rcopies:
            c.wait()
        lax.fori_loop(0, _NCH // 2, step, 0)
        drain(0)
        pltpu.sync_copy(out_v, out_hbm.at[pl.ds(base + half * _HALF, _HALF)])


def kernel(entity_embedding, relation_embedding, heads, relations, tails):
    mesh = plsc.VectorSubcoreMesh(core_axis_name="c", subcore_axis_name="s")
    run = pl.kernel(
        _body,
        out_type=jax.ShapeDtypeStruct((_B,), jnp.float32),
        mesh=mesh,
        scratch_types=[
            pltpu.VMEM((_BPW,), jnp.int32),
            pltpu.VMEM((_BPW,), jnp.int32),
            pltpu.VMEM((_BPW,), jnp.int32),
            pltpu.VMEM((_BPW,), jnp.int32),
            pltpu.VMEM((2, _CH, 1, _D), jnp.float32),
            pltpu.VMEM((2, _CH, 1, _D), jnp.float32),
            pltpu.VMEM((_HALF, 2 * _D), jnp.float32),
            pltpu.VMEM((_HALF,), jnp.float32),
            pltpu.SemaphoreType.DMA,
            pltpu.SemaphoreType.DMA,
        ],
    )
    return run(entity_embedding, relation_embedding.reshape(_NR // 2, 128),
               heads.astype(jnp.int32), relations.astype(jnp.int32),
               tails.astype(jnp.int32))


# speedup vs baseline: 4.5762x; 1.0036x over previous
import jax
import jax.numpy as jnp
from jax import lax
from jax.experimental import pallas as pl
from jax.experimental.pallas import tpu as pltpu
from jax.experimental.pallas import tpu_sc as plsc

_B = 16384
_D = 64
_NR = 1000
_NC = 2
_NS = 16
_NW = _NC * _NS
_BPW = _B // _NW
_HALF = _BPW // 2
_CH = 16
_NCH = _HALF // _CH


def _fire(ent_hbm, hvec, tvec, e1b, e2b, sem):
    for j in range(_CH):
        pltpu.async_copy(ent_hbm.at[pl.ds(hvec[j], 1), :], e1b.at[j], sem)
        pltpu.async_copy(ent_hbm.at[pl.ds(tvec[j], 1), :], e2b.at[j], sem)


def _body(ent_hbm, rel_hbm, heads_hbm, rels_hbm, tails_hbm, out_hbm,
          hidx, ridx, tidx, rp, e1b2, e2b2, r_v, out_v, sem, rsem):
    wid = lax.axis_index("s") * _NC + lax.axis_index("c")
    base = wid * _BPW

    pltpu.sync_copy(heads_hbm.at[pl.ds(base, _BPW)], hidx)
    pltpu.sync_copy(rels_hbm.at[pl.ds(base, _BPW)], ridx)
    pltpu.sync_copy(tails_hbm.at[pl.ds(base, _BPW)], tidx)

    for k in range(_BPW // 16):
        sl = pl.ds(k * 16, 16)
        rp[sl] = ridx[sl] >> 1

    lanes16 = lax.iota(jnp.int32, 16)
    ones16 = jnp.ones((16,), jnp.int32)
    bfly = [jnp.bitwise_xor(lanes16, sh) for sh in (8, 4, 2, 1)]
    dnums = lax.GatherDimensionNumbers(
        offset_dims=(), collapsed_slice_dims=(0,), start_index_map=(0,))

    def shuffle(v, idx):
        return lax.gather(v, idx[:, None], dnums, slice_sizes=(1,),
                          mode=lax.GatherScatterMode.PROMISE_IN_BOUNDS)

    def lanesum(v):
        for idx in bfly:
            v = v + shuffle(v, idx)
        return v

    def idx_chunk(half, k):
        sl = pl.ds(half * _HALF + k * _CH, _CH)
        return hidx[sl], tidx[sl]

    def drain(slot):
        dummy = ent_hbm.at[pl.ds(0, _CH), :]
        pltpu.make_async_copy(dummy, e1b2.at[slot, :, 0, :], sem).wait()
        pltpu.make_async_copy(dummy, e2b2.at[slot, :, 0, :], sem).wait()

    def fire_chunk(half, k, slot):
        hv, tv = idx_chunk(half, k)
        _fire(ent_hbm, hv, tv, e1b2.at[slot], e2b2.at[slot], sem)

    def compute_chunk(half, k, slot):
        sl = pl.ds(half * _HALF + k * _CH, _CH)
        rparf = (ridx[sl] & ones16).astype(jnp.float32)
        s = jnp.zeros((16,), jnp.float32)
        for j in range(_CH):
            row = k * _CH + j
            jv = jnp.full((16,), j, jnp.int32)
            pr = shuffle(rparf, jv)
            acc = jnp.zeros((16,), jnp.float32)
            for c in range(_D // 16):
                lo = pl.ds(c * 16, 16)
                hi = pl.ds(64 + c * 16, 16)
                b1 = r_v[row, lo]
                b = b1 + pr * (r_v[row, hi] - b1)
                acc = acc + (e1b2[slot, j, 0, lo] * b) \
                    * e2b2[slot, j, 0, lo]
            s = jnp.where(lanes16 == j, lanesum(acc), s)
        out_v[pl.ds(k * _CH, _CH)] = 1.0 / (1.0 + jnp.exp(-s))

    for half in range(2):
        rcopies = []
        for q in range(_HALF // 128):
            isl = pl.ds(half * _HALF + q * 128, 128)
            vsl = pl.ds(q * 128, 128)
            rcopies.append(
                pltpu.async_copy(rel_hbm.at[rp.at[isl]], r_v.at[vsl], rsem))

        fire_chunk(half, 0, 0)

        def step(i, carry):
            a = i * 2
            fire_chunk(half, a + 1, 1)
            drain(0)
            compute_chunk(half, a, 0)
            fire_chunk(half, jnp.minimum(a + 2, _NCH - 1), 0)
            drain(1)
            compute_chunk(half, a + 1, 1)
            return carry

        for c in rcopies:
            c.wait()
        lax.fori_loop(0, _NCH // 2, step, 0)
        drain(0)
        pltpu.sync_copy(out_v, out_hbm.at[pl.ds(base + half * _HALF, _HALF)])


def kernel(entity_embedding, relation_embedding, heads, relations, tails):
    mesh = plsc.VectorSubcoreMesh(core_axis_name="c", subcore_axis_name="s")
    run = pl.kernel(
        _body,
        out_type=jax.ShapeDtypeStruct((_B,), jnp.float32),
        mesh=mesh,
        scratch_types=[
            pltpu.VMEM((_BPW,), jnp.int32),
            pltpu.VMEM((_BPW,), jnp.int32),
            pltpu.VMEM((_BPW,), jnp.int32),
            pltpu.VMEM((_BPW,), jnp.int32),
            pltpu.VMEM((2, _CH, 1, _D), jnp.float32),
            pltpu.VMEM((2, _CH, 1, _D), jnp.float32),
            pltpu.VMEM((_HALF, 2 * _D), jnp.float32),
            pltpu.VMEM((_HALF,), jnp.float32),
            pltpu.SemaphoreType.DMA,
            pltpu.SemaphoreType.DMA,
        ],
    )
    return run(entity_embedding, relation_embedding.reshape(_NR // 2, 128),
               heads.astype(jnp.int32), relations.astype(jnp.int32),
               tails.astype(jnp.int32))
